# baseline (speedup 1.0000x reference)
import functools

import jax
import jax.numpy as jnp
from jax import lax
from jax.experimental import pallas as pl
from jax.experimental.pallas import tpu as pltpu
from jax.experimental.pallas import tpu_sc as plsc

N_EXPERTS = 64
TOP_K = 8
N_GROUP = 8
GROUP_SIZE = N_EXPERTS // N_GROUP
TOPK_GROUP = 4
ROUTED_SCALING_FACTOR = 2.5
_NEG = -1e30

L = 16
NW = 32


def _sfc_kernel(x_ref, wt_ref, bias_ref, out_ref):
    logits = jnp.dot(x_ref[...], wt_ref[...], preferred_element_type=jnp.float32)
    out_ref[...] = jax.nn.sigmoid(logits.T) + bias_ref[...]


def _tc_sfc(flat, wt, bias, token_block):
    t = flat.shape[0]
    return pl.pallas_call(
        _sfc_kernel,
        grid=(t // token_block,),
        in_specs=[
            pl.BlockSpec((token_block, flat.shape[1]), lambda i: (i, 0)),
            pl.BlockSpec((flat.shape[1], N_EXPERTS), lambda i: (0, 0)),
            pl.BlockSpec((N_EXPERTS, 1), lambda i: (0, 0)),
        ],
        out_specs=pl.BlockSpec((N_EXPERTS, token_block), lambda i: (0, i)),
        out_shape=jax.ShapeDtypeStruct((N_EXPERTS, t), jnp.float32),
    )(flat, wt, bias)


def _tree_argmax(vals):
    nodes = [(v, jnp.full((L,), i, jnp.int32)) for i, v in enumerate(vals)]
    while len(nodes) > 1:
        nxt = []
        for i in range(0, len(nodes), 2):
            (av, ai), (bv, bi) = nodes[i], nodes[i + 1]
            c = av >= bv
            nxt.append((jnp.where(c, av, bv), jnp.where(c, ai, bi)))
        nodes = nxt
    return nodes[0]


def _merge_top2(a, b):
    hi = jnp.maximum(a[0], b[0])
    lo = jnp.maximum(jnp.minimum(a[0], b[0]), jnp.maximum(a[1], b[1]))
    return hi, lo


def _sc_route(sfc_t, bias1d, tok_w):
    t = sfc_t.shape[1]
    tiles = tok_w // L

    @functools.partial(
        pl.kernel,
        mesh=plsc.VectorSubcoreMesh(core_axis_name="c", subcore_axis_name="s"),
        out_type=[
            jax.ShapeDtypeStruct((TOP_K, t), jnp.int32),
            jax.ShapeDtypeStruct((TOP_K, t), jnp.float32),
        ],
        scratch_types=[
            pltpu.VMEM((N_EXPERTS, tok_w), jnp.float32),
            pltpu.VMEM((N_EXPERTS // L, L), jnp.float32),
            pltpu.VMEM((TOP_K, tok_w), jnp.int32),
            pltpu.VMEM((TOP_K, tok_w), jnp.float32),
        ],
    )
    def route(sfc_hbm, bias_hbm, idx_hbm, w_hbm, slab, bias_v, idx_s, w_s):
        wid = lax.axis_index("s") * 2 + lax.axis_index("c")
        base = wid * tok_w
        pltpu.sync_copy(sfc_hbm.at[:, pl.ds(base, tok_w)], slab)
        pltpu.sync_copy(bias_hbm, bias_v)
        bias_regs = [bias_v[j, :] for j in range(N_EXPERTS // L)]

        def tile_body(ti, carry):
            o = ti * L
            sfc = [slab[e, pl.ds(o, L)] for e in range(N_EXPERTS)]

            gsum = []
            for g in range(N_GROUP):
                v = sfc[g * GROUP_SIZE : (g + 1) * GROUP_SIZE]
                p = [
                    (jnp.maximum(v[2 * i], v[2 * i + 1]), jnp.minimum(v[2 * i], v[2 * i + 1]))
                    for i in range(GROUP_SIZE // 2)
                ]
                hi, lo = _merge_top2(_merge_top2(p[0], p[1]), _merge_top2(p[2], p[3]))
                gsum.append(hi + lo)

            picked = []
            for _ in range(TOPK_GROUP):
                _, gi = _tree_argmax(gsum)
                picked.append(gi)
                gsum = [jnp.where(gi == g, _NEG, gsum[g]) for g in range(N_GROUP)]
            kg = [
                (picked[0] == g) | (picked[1] == g) | (picked[2] == g) | (picked[3] == g)
                for g in range(N_GROUP)
            ]

            mk = [
                jnp.where(kg[e // GROUP_SIZE], sfc[e], 0.0) for e in range(N_EXPERTS)
            ]
            wks = []
            for k in range(TOP_K):
                mv, ei = _tree_argmax(mk)
                lo = jnp.bitwise_and(ei, L - 1)
                hi4 = jax.lax.shift_right_logical(ei, 4)
                bk = jnp.zeros((L,), jnp.float32)
                dnums = lax.GatherDimensionNumbers(
                    offset_dims=(), collapsed_slice_dims=(0,), start_index_map=(0,)
                )
                for j in range(N_EXPERTS // L):
                    cand = lax.gather(
                        bias_regs[j],
                        lo[:, None],
                        dnums,
                        (1,),
                        mode=lax.GatherScatterMode.PROMISE_IN_BOUNDS,
                    )
                    bk = jnp.where(hi4 == j, cand, bk)
                wks.append(mv - bk)
                idx_s[k, pl.ds(o, L)] = ei
                mk = [jnp.where(ei == e, _NEG, mk[e]) for e in range(N_EXPERTS)]
            denom = wks[0]
            for k in range(1, TOP_K):
                denom = denom + wks[k]
            scale = ROUTED_SCALING_FACTOR / (denom + 1e-20)
            for k in range(TOP_K):
                w_s[k, pl.ds(o, L)] = wks[k] * scale
            return carry

        lax.fori_loop(0, tiles, tile_body, 0)
        pltpu.sync_copy(idx_s, idx_hbm.at[:, pl.ds(base, tok_w)])
        pltpu.sync_copy(w_s, w_hbm.at[:, pl.ds(base, tok_w)])

    return route(sfc_t, bias1d)


@jax.jit
def _hybrid(flat, wt, bias_col, bias1d):
    sfc_t = _tc_sfc(flat, wt, bias_col, 1024)
    return _sc_route(sfc_t, bias1d, flat.shape[0] // NW)


def kernel(hidden_states, weight, e_score_correction_bias):
    bsz, seq_len, hidden_dim = hidden_states.shape
    flat = hidden_states.reshape(-1, hidden_dim).astype(jnp.float32)
    wt = weight.astype(jnp.float32).T
    bias1d = e_score_correction_bias.astype(jnp.float32)
    idx_t, w_t = _hybrid(
        flat, wt, bias1d.reshape(N_EXPERTS, 1), bias1d.reshape(N_EXPERTS // L, L)
    )
    return idx_t.T, w_t.T


# speedup vs baseline: 1.4945x; 1.4945x over previous
import functools

import jax
import jax.numpy as jnp
from jax.experimental import pallas as pl
from jax.experimental.pallas import tpu as pltpu

N_EXPERTS = 64
TOP_K = 8
N_GROUP = 8
GROUP_SIZE = N_EXPERTS // N_GROUP
TOPK_GROUP = 4
ROUTED_SCALING_FACTOR = 2.5

_NEG = -1e30


def _router_kernel(x_ref, wt_ref, bias_ref, idx_ref, w_ref):
    x = x_ref[...]
    logits = jnp.dot(x, wt_ref[...], preferred_element_type=jnp.float32)
    lt = logits.T
    tb = lt.shape[1]
    scores = jax.nn.sigmoid(lt)
    sfc = scores + bias_ref[...]

    sub = jax.lax.broadcasted_iota(jnp.int32, (N_EXPERTS, tb), 0)

    gi8 = jax.lax.broadcasted_iota(jnp.int32, (GROUP_SIZE, tb), 0)
    gs = []
    for g in range(N_GROUP):
        sg = jax.lax.slice_in_dim(sfc, g * GROUP_SIZE, (g + 1) * GROUP_SIZE, axis=0)
        m1 = jnp.max(sg, axis=0, keepdims=True)
        fi = jnp.min(jnp.where(sg == m1, gi8, GROUP_SIZE), axis=0, keepdims=True)
        m2 = jnp.max(jnp.where(gi8 == fi, _NEG, sg), axis=0, keepdims=True)
        gs.append(m1 + m2)
    gsum = jnp.concatenate(gs, axis=0)

    gi = jax.lax.broadcasted_iota(jnp.int32, (N_GROUP, tb), 0)
    keep8 = jnp.zeros((N_GROUP, tb), dtype=jnp.bool_)
    for _ in range(TOPK_GROUP):
        gm = jnp.max(gsum, axis=0, keepdims=True)
        fi = jnp.min(jnp.where(gsum == gm, gi, N_GROUP), axis=0, keepdims=True)
        pick = gi == fi
        keep8 = jnp.logical_or(keep8, pick)
        gsum = jnp.where(pick, _NEG, gsum)
    keep = jnp.concatenate(
        [jnp.broadcast_to(keep8[g : g + 1, :], (GROUP_SIZE, tb)) for g in range(N_GROUP)],
        axis=0,
    )

    masked = jnp.where(keep, sfc, 0.0)
    idxs, ws = [], []
    for _ in range(TOP_K):
        m = jnp.max(masked, axis=0, keepdims=True)
        fi = jnp.min(jnp.where(masked == m, sub, N_EXPERTS), axis=0, keepdims=True)
        hit = sub == fi
        wv = jnp.sum(jnp.where(hit, scores, 0.0), axis=0, keepdims=True)
        masked = jnp.where(hit, _NEG, masked)
        idxs.append(fi)
        ws.append(wv)
    idx_t = jnp.concatenate(idxs, axis=0)
    w_t = jnp.concatenate(ws, axis=0)
    denom = jnp.sum(w_t, axis=0, keepdims=True) + 1e-20
    idx_ref[...] = idx_t
    w_ref[...] = w_t * (ROUTED_SCALING_FACTOR / denom)


@functools.partial(jax.jit, static_argnames=("token_block",))
def _route(flat, wt, bias, token_block):
    t = flat.shape[0]
    grid = (t // token_block,)
    return pl.pallas_call(
        _router_kernel,
        grid=grid,
        in_specs=[
            pl.BlockSpec((token_block, flat.shape[1]), lambda i: (i, 0)),
            pl.BlockSpec((flat.shape[1], N_EXPERTS), lambda i: (0, 0)),
            pl.BlockSpec((N_EXPERTS, 1), lambda i: (0, 0)),
        ],
        out_specs=[
            pl.BlockSpec((TOP_K, token_block), lambda i: (0, i)),
            pl.BlockSpec((TOP_K, token_block), lambda i: (0, i)),
        ],
        out_shape=[
            jax.ShapeDtypeStruct((TOP_K, t), jnp.int32),
            jax.ShapeDtypeStruct((TOP_K, t), jnp.float32),
        ],
    )(flat, wt, bias)


def kernel(hidden_states, weight, e_score_correction_bias):
    bsz, seq_len, hidden_dim = hidden_states.shape
    flat = hidden_states.reshape(-1, hidden_dim).astype(jnp.float32)
    wt = weight.astype(jnp.float32).T
    bias = e_score_correction_bias.astype(jnp.float32).reshape(N_EXPERTS, 1)
    idx_t, w_t = _route(flat, wt, bias, token_block=1024)
    return idx_t.T, w_t.T


# speedup vs baseline: 1.4974x; 1.0019x over previous
import functools

import jax
import jax.numpy as jnp
from jax.experimental import pallas as pl

N_EXPERTS = 64
TOP_K = 8
N_GROUP = 8
GROUP_SIZE = N_EXPERTS // N_GROUP
TOPK_GROUP = 4
ROUTED_SCALING_FACTOR = 2.5

_NEG = -1e30


def _router_kernel(x_ref, wt_ref, bias_ref, idx_ref, w_ref):
    x = x_ref[...]
    logits = jnp.dot(x, wt_ref[...], preferred_element_type=jnp.float32)
    lt = logits.T
    tb = lt.shape[1]
    scores = jax.nn.sigmoid(lt)
    sfc = scores + bias_ref[...]

    sub = jax.lax.broadcasted_iota(jnp.int32, (N_EXPERTS, tb), 0)

    gi8 = jax.lax.broadcasted_iota(jnp.int32, (GROUP_SIZE, tb), 0)
    gs = []
    for g in range(N_GROUP):
        sg = jax.lax.slice_in_dim(sfc, g * GROUP_SIZE, (g + 1) * GROUP_SIZE, axis=0)
        m1 = jnp.max(sg, axis=0, keepdims=True)
        fi = jnp.min(jnp.where(sg == m1, gi8, GROUP_SIZE), axis=0, keepdims=True)
        m2 = jnp.max(jnp.where(gi8 == fi, _NEG, sg), axis=0, keepdims=True)
        gs.append(m1 + m2)
    gsum = jnp.concatenate(gs, axis=0)

    gi = jax.lax.broadcasted_iota(jnp.int32, (N_GROUP, tb), 0)
    keep8 = jnp.zeros((N_GROUP, tb), dtype=jnp.bool_)
    for _ in range(TOPK_GROUP):
        gm = jnp.max(gsum, axis=0, keepdims=True)
        fi = jnp.min(jnp.where(gsum == gm, gi, N_GROUP), axis=0, keepdims=True)
        pick = gi == fi
        keep8 = jnp.logical_or(keep8, pick)
        gsum = jnp.where(pick, _NEG, gsum)
    keep = jnp.concatenate(
        [jnp.broadcast_to(keep8[g : g + 1, :], (GROUP_SIZE, tb)) for g in range(N_GROUP)],
        axis=0,
    )

    masked = jnp.where(keep, sfc, 0.0)
    idxs, ws = [], []
    for _ in range(TOP_K):
        m = jnp.max(masked, axis=0, keepdims=True)
        fi = jnp.min(jnp.where(masked == m, sub, N_EXPERTS), axis=0, keepdims=True)
        hit = sub == fi
        wv = jnp.sum(jnp.where(hit, scores, 0.0), axis=0, keepdims=True)
        masked = jnp.where(hit, _NEG, masked)
        idxs.append(fi)
        ws.append(wv)
    idx_t = jnp.concatenate(idxs, axis=0)
    w_t = jnp.concatenate(ws, axis=0)
    denom = jnp.sum(w_t, axis=0, keepdims=True) + 1e-20
    idx_ref[...] = idx_t
    w_ref[...] = w_t * (ROUTED_SCALING_FACTOR / denom)


@functools.partial(jax.jit, static_argnames=("token_block",))
def _route(flat, wt, bias, token_block):
    t = flat.shape[0]
    grid = (t // token_block,)
    return pl.pallas_call(
        _router_kernel,
        grid=grid,
        in_specs=[
            pl.BlockSpec((token_block, flat.shape[1]), lambda i: (i, 0)),
            pl.BlockSpec((flat.shape[1], N_EXPERTS), lambda i: (0, 0)),
            pl.BlockSpec((N_EXPERTS, 1), lambda i: (0, 0)),
        ],
        out_specs=[
            pl.BlockSpec((TOP_K, token_block), lambda i: (0, i)),
            pl.BlockSpec((TOP_K, token_block), lambda i: (0, i)),
        ],
        out_shape=[
            jax.ShapeDtypeStruct((TOP_K, t), jnp.int32),
            jax.ShapeDtypeStruct((TOP_K, t), jnp.float32),
        ],
    )(flat, wt, bias)


def kernel(hidden_states, weight, e_score_correction_bias):
    bsz, seq_len, hidden_dim = hidden_states.shape
    flat = hidden_states.reshape(-1, hidden_dim).astype(jnp.float32)
    wt = weight.astype(jnp.float32).T
    bias = e_score_correction_bias.astype(jnp.float32).reshape(N_EXPERTS, 1)
    idx_t, w_t = _route(flat, wt, bias, token_block=1024)
    return idx_t.T, w_t.T
